# Initial kernel scaffold; baseline (speedup 1.0000x reference)
import jax
import jax.numpy as jnp
from jax.experimental import pallas as pl


def kernel(x, edge_index, edge_attr, batch, node_table, edge_table, ee_W, ee_b, pre_W, pre_b, post_W, post_b, lin_W, lin_b, bn_g, bn_b, m1W, m1b, m2W, m2b, m3W, m3b):
    raise NotImplementedError("write your pallas kernel here")


# speedup vs baseline: 47.0605x; 47.0605x over previous
import functools

import jax
import jax.numpy as jnp
import numpy as np
from jax import lax
from jax.experimental import pallas as pl
from jax.experimental.pallas import tpu as pltpu
from jax.experimental.pallas import tpu_sc as plsc

N_NODES = 10000
N_EDGES = 160000
N_GRAPHS = 256
NB = 32
NBLK = 313
NP = NBLK * NB
NWORK = 32
BPW = 10
K = 128
EPAD = N_EDGES + K
F = 375
FP = 384
NSL = FP // 16
RB = 1000
NRB = N_NODES // RB
OFFS_PAD = 336

_DEG_HIST = np.array([0, 0, 1, 3, 10, 26, 60, 120, 211, 331, 473, 620, 744,
                      826, 862, 855, 806, 724, 621, 510, 403, 306, 224, 158,
                      107, 70, 44, 27, 16, 9, 5, 3, 1], dtype=np.float64)
AVG_DEG_LOG = float((np.log(np.arange(len(_DEG_HIST)) + 1.0) * _DEG_HIST).sum()
                    / _DEG_HIST.sum())
HI = lax.Precision.HIGHEST
f32 = jnp.float32


def _dot(a, b):
    return jnp.dot(a, b, preferred_element_type=f32, precision=HI)


def _embed_body(x_ref, nt_ref, h_ref):
    xv = x_ref[0, 0, :]
    oh = (xv[:, None] == lax.broadcasted_iota(jnp.int32, (RB, 128), 1)
          ).astype(f32)
    h_ref[...] = _dot(oh, nt_ref[...])


def _embed(x3, node_table):
    return pl.pallas_call(
        _embed_body,
        grid=(NRB,),
        in_specs=[pl.BlockSpec((1, 1, RB), lambda i: (i, 0, 0)),
                  pl.BlockSpec((128, 75), lambda i: (0, 0))],
        out_specs=pl.BlockSpec((RB, 75), lambda i: (i, 0)),
        out_shape=jax.ShapeDtypeStruct((N_NODES, 75), f32),
    )(x3, node_table)


def _project_body(h_ref, wd_ref, ws_ref, et_ref, eew_ref, eeb_ref, we_ref,
                  preb_ref, a_ref, b_ref, c_ref):
    h = h_ref[...]
    a_ref[...] = _dot(h, wd_ref[...])
    b_ref[...] = _dot(h, ws_ref[...])

    @pl.when(pl.program_id(0) == 0)
    def _():
        eenc = _dot(et_ref[...], eew_ref[...]) + eeb_ref[...]
        c_ref[...] = _dot(eenc, we_ref[...]) + preb_ref[...]


def _project(h, wd, ws, edge_table, eew, eeb, we, preb):
    return pl.pallas_call(
        _project_body,
        grid=(NRB,),
        in_specs=[pl.BlockSpec((RB, 75), lambda i: (i, 0)),
                  pl.BlockSpec((75, FP), lambda i: (0, 0)),
                  pl.BlockSpec((75, FP), lambda i: (0, 0)),
                  pl.BlockSpec((16, 50), lambda i: (0, 0)),
                  pl.BlockSpec((50, 75), lambda i: (0, 0)),
                  pl.BlockSpec((1, 75), lambda i: (0, 0)),
                  pl.BlockSpec((75, FP), lambda i: (0, 0)),
                  pl.BlockSpec((1, FP), lambda i: (0, 0))],
        out_specs=[pl.BlockSpec((RB, FP), lambda i: (i, 0)),
                   pl.BlockSpec((RB, FP), lambda i: (i, 0)),
                   pl.BlockSpec((16, FP), lambda i: (0, 0))],
        out_shape=[jax.ShapeDtypeStruct((N_NODES, FP), f32),
                   jax.ShapeDtypeStruct((N_NODES, FP), f32),
                   jax.ShapeDtypeStruct((16, FP), f32)],
    )(h, wd, ws, edge_table, eew, eeb, we, preb)


def _scalar_at(ref, i):
    return ref[pl.ds(i, 16)][0]


def _seg_body(b_hbm, c_hbm, keys_hbm, offs_hbm, s1_hbm, s2_hbm, mn_hbm,
              mx_hbm, deg_hbm, b_rows, kbuf, src_buf, dst_buf, attr_buf,
              c_loc, acc_s1, acc_s2, acc_mn, acc_mx, acc_dg, offs_loc, sem):
    wid = lax.axis_index("s") + 16 * lax.axis_index("c")
    pltpu.sync_copy(offs_hbm, offs_loc)
    pltpu.sync_copy(c_hbm, c_loc)

    lane = lax.iota(jnp.int32, 16)
    zero16 = jnp.zeros((16,), f32)
    inf16 = jnp.full((16,), jnp.inf, f32)
    ninf16 = jnp.full((16,), -jnp.inf, f32)
    one0 = jnp.where(lane == 0, 1.0, 0.0).astype(f32)

    def blk_loop(i, carry):
        blk = wid + NWORK * i

        @pl.when(blk < NBLK)
        def _():
            e0 = _scalar_at(offs_loc, blk)
            e1 = _scalar_at(offs_loc, blk + 1)
            node0 = blk * NB

            def init_row(r, c2):
                for c in range(NSL):
                    sl = pl.ds(c * 16, 16)
                    acc_s1[r, sl] = zero16
                    acc_s2[r, sl] = zero16
                    acc_mn[r, sl] = inf16
                    acc_mx[r, sl] = ninf16
                acc_dg[r, pl.ds(0, 16)] = zero16
                return c2
            lax.fori_loop(0, NB, init_row, 0)

            al0 = (e0 >> 3) << 3
            nch = (e1 - al0 + (K - 1)) >> 7

            def ch_loop(ci, c2):
                eb = pl.multiple_of(al0 + ci * K, 8)
                pltpu.sync_copy(keys_hbm.at[pl.ds(eb, K)], kbuf)
                for v in range(K // 16):
                    sl = pl.ds(v * 16, 16)
                    kv = kbuf[sl]
                    dst_buf[sl] = (kv >> jnp.uint32(18)).astype(jnp.int32)
                    src_buf[sl] = ((kv >> jnp.uint32(4))
                                   & jnp.uint32(0x3FFF)).astype(jnp.int32)
                    attr_buf[sl] = (kv & jnp.uint32(0xF)).astype(jnp.int32)
                pltpu.async_copy(b_hbm.at[src_buf], b_rows, sem).wait()
                lo = jnp.maximum(e0, eb)
                hi = jnp.minimum(e1, eb + K)

                def e_loop(j, c3):
                    jl = j - eb
                    dl = _scalar_at(dst_buf, jl) - node0
                    av = _scalar_at(attr_buf, jl)
                    for c in range(NSL):
                        sl = pl.ds(c * 16, 16)
                        t = b_rows[jl, sl] + c_loc[av, sl]
                        plsc.addupdate(acc_s1.at[dl, sl], t)
                        plsc.addupdate(acc_s2.at[dl, sl], t * t)
                        acc_mn[dl, sl] = jnp.minimum(acc_mn[dl, sl], t)
                        acc_mx[dl, sl] = jnp.maximum(acc_mx[dl, sl], t)
                    plsc.addupdate(acc_dg.at[dl, pl.ds(0, 16)], one0)
                    return c3
                lax.fori_loop(lo, hi, e_loop, 0)
                return c2
            lax.fori_loop(0, nch, ch_loop, 0)

            pltpu.sync_copy(acc_s1, s1_hbm.at[pl.ds(node0, NB)])
            pltpu.sync_copy(acc_s2, s2_hbm.at[pl.ds(node0, NB)])
            pltpu.sync_copy(acc_mn, mn_hbm.at[pl.ds(node0, NB)])
            pltpu.sync_copy(acc_mx, mx_hbm.at[pl.ds(node0, NB)])
            pltpu.sync_copy(acc_dg, deg_hbm.at[pl.ds(node0, NB)])
        return carry

    lax.fori_loop(0, BPW, blk_loop, 0)


@functools.cache
def _build_segreduce():
  return pl.kernel(
    _seg_body,
    mesh=plsc.VectorSubcoreMesh(core_axis_name="c", subcore_axis_name="s"),
    out_type=[jax.ShapeDtypeStruct((NP, FP), f32),
              jax.ShapeDtypeStruct((NP, FP), f32),
              jax.ShapeDtypeStruct((NP, FP), f32),
              jax.ShapeDtypeStruct((NP, FP), f32),
              jax.ShapeDtypeStruct((NP, 16), f32)],
    scratch_types=[pltpu.VMEM((K, FP), f32),
                   pltpu.VMEM((K,), jnp.uint32),
                   pltpu.VMEM((K,), jnp.int32),
                   pltpu.VMEM((K + 16,), jnp.int32),
                   pltpu.VMEM((K + 16,), jnp.int32),
                   pltpu.VMEM((16, FP), f32),
                   pltpu.VMEM((NB, FP), f32),
                   pltpu.VMEM((NB, FP), f32),
                   pltpu.VMEM((NB, FP), f32),
                   pltpu.VMEM((NB, FP), f32),
                   pltpu.VMEM((NB, 16), f32),
                   pltpu.VMEM((OFFS_PAD,), jnp.int32),
                   pltpu.SemaphoreType.DMA])


def _segreduce(bm, c, ksp, offs):
    return _build_segreduce()(bm, c, ksp, offs)


def _final_body(h_ref, a_ref, s1_ref, s2_ref, mn_ref, mx_ref, deg_ref,
                pw_ref, pb_ref, lw_ref, lb_ref, pre_ref, st_ref):
    deg_raw = deg_ref[:, 0:1]
    has = deg_raw > 0
    degc = jnp.maximum(deg_raw, 1.0)
    a = a_ref[:, :F]
    s1d = s1_ref[:, :F] / degc
    mean = jnp.where(has, a + s1d, 0.0)
    sqm = jnp.where(has, a * a + 2.0 * a * s1d + s2_ref[:, :F] / degc, 0.0)
    std = jnp.sqrt(jax.nn.relu(sqm - mean * mean) + 1e-5)
    mn = jnp.where(has, a + mn_ref[:, :F], 0.0)
    mx = jnp.where(has, a + mx_ref[:, :F], 0.0)
    amp = jnp.log(degc + 1.0) * (1.0 / AVG_DEG_LOG)
    iamp = 1.0 / amp
    h = h_ref[...]
    ys = []
    for t in range(5):
        sl = slice(t * 75, (t + 1) * 75)
        parts = [mean[:, sl], mn[:, sl], mx[:, sl], std[:, sl]]
        feats = jnp.concatenate(
            [h] + parts + [p * amp for p in parts] + [p * iamp for p in parts],
            axis=1)
        ys.append(_dot(feats, pw_ref[t]) + pb_ref[t, :][None, :])
    out75 = jnp.concatenate(ys, axis=1)
    pre = _dot(out75, lw_ref[...]) + lb_ref[...]
    pre_ref[...] = pre

    @pl.when(pl.program_id(0) == 0)
    def _():
        st_ref[...] = jnp.zeros((8, 128), f32)

    st_ref[0:1, :75] = st_ref[0:1, :75] + jnp.sum(pre, axis=0)[None, :]
    st_ref[1:2, :75] = st_ref[1:2, :75] + jnp.sum(pre * pre, axis=0)[None, :]


def _finalize(h, a, s1, s2, mn, mx, deg, pw, pb, lw, lb):
    return pl.pallas_call(
        _final_body,
        grid=(NRB,),
        in_specs=[pl.BlockSpec((RB, 75), lambda i: (i, 0)),
                  pl.BlockSpec((RB, FP), lambda i: (i, 0)),
                  pl.BlockSpec((RB, FP), lambda i: (i, 0)),
                  pl.BlockSpec((RB, FP), lambda i: (i, 0)),
                  pl.BlockSpec((RB, FP), lambda i: (i, 0)),
                  pl.BlockSpec((RB, FP), lambda i: (i, 0)),
                  pl.BlockSpec((RB, 16), lambda i: (i, 0)),
                  pl.BlockSpec((5, 975, 15), lambda i: (0, 0, 0)),
                  pl.BlockSpec((5, 15), lambda i: (0, 0)),
                  pl.BlockSpec((75, 75), lambda i: (0, 0)),
                  pl.BlockSpec((1, 75), lambda i: (0, 0))],
        out_specs=[pl.BlockSpec((RB, 75), lambda i: (i, 0)),
                   pl.BlockSpec((8, 128), lambda i: (0, 0))],
        out_shape=[jax.ShapeDtypeStruct((N_NODES, 75), f32),
                   jax.ShapeDtypeStruct((8, 128), f32)],
    )(h, a, s1, s2, mn, mx, deg, pw, pb, lw, lb)


def _bn_body(pre_ref, st_ref, g_ref, b_ref, h_ref):
    mu = st_ref[0:1, :75] * (1.0 / N_NODES)
    ex2 = st_ref[1:2, :75] * (1.0 / N_NODES)
    var = ex2 - mu * mu
    scale = g_ref[...] / jnp.sqrt(var + 1e-5)
    h_ref[...] = jax.nn.relu((pre_ref[...] - mu) * scale + b_ref[...])


def _bn(pre, st, g, b):
    return pl.pallas_call(
        _bn_body,
        grid=(NRB,),
        in_specs=[pl.BlockSpec((RB, 75), lambda i: (i, 0)),
                  pl.BlockSpec((8, 128), lambda i: (0, 0)),
                  pl.BlockSpec((1, 75), lambda i: (0, 0)),
                  pl.BlockSpec((1, 75), lambda i: (0, 0))],
        out_specs=pl.BlockSpec((RB, 75), lambda i: (i, 0)),
        out_shape=jax.ShapeDtypeStruct((N_NODES, 75), f32),
    )(pre, st, g, b)


def _pool_body(b3_ref, h_ref, w1_ref, b1_ref, w2_ref, b2_ref, w3_ref, b3b_ref,
               out_ref, g_ref):
    i = pl.program_id(0)

    @pl.when(i == 0)
    def _():
        g_ref[...] = jnp.zeros((N_GRAPHS, 75), f32)

    bv = b3_ref[0, 0, :]
    oh = (bv[:, None] == lax.broadcasted_iota(jnp.int32, (RB, N_GRAPHS), 1)
          ).astype(f32)
    g_ref[...] = g_ref[...] + lax.dot_general(
        oh, h_ref[...], (((0,), (0,)), ((), ())),
        preferred_element_type=f32, precision=HI)

    @pl.when(i == NRB - 1)
    def _():
        z = jax.nn.relu(_dot(g_ref[...], w1_ref[...]) + b1_ref[...])
        z = jax.nn.relu(_dot(z, w2_ref[...]) + b2_ref[...])
        out_ref[...] = _dot(z, w3_ref[...]) + b3b_ref[...]


def _pool(batch3, h, w1, b1, w2, b2, w3, b3):
    return pl.pallas_call(
        _pool_body,
        grid=(NRB,),
        in_specs=[pl.BlockSpec((1, 1, RB), lambda i: (i, 0, 0)),
                  pl.BlockSpec((RB, 75), lambda i: (i, 0)),
                  pl.BlockSpec((75, 50), lambda i: (0, 0)),
                  pl.BlockSpec((1, 50), lambda i: (0, 0)),
                  pl.BlockSpec((50, 25), lambda i: (0, 0)),
                  pl.BlockSpec((1, 25), lambda i: (0, 0)),
                  pl.BlockSpec((25, 10), lambda i: (0, 0)),
                  pl.BlockSpec((1, 10), lambda i: (0, 0))],
        out_specs=pl.BlockSpec((N_GRAPHS, 10), lambda i: (0, 0)),
        out_shape=jax.ShapeDtypeStruct((N_GRAPHS, 10), f32),
        scratch_shapes=[pltpu.VMEM((N_GRAPHS, 75), f32)],
    )(batch3, h, w1, b1, w2, b2, w3, b3)


def kernel(x, edge_index, edge_attr, batch, node_table, edge_table, ee_W,
           ee_b, pre_W, pre_b, post_W, post_b, lin_W, lin_b, bn_g, bn_b,
           m1W, m1b, m2W, m2b, m3W, m3b):
    src = edge_index[0].astype(jnp.uint32)
    dst = edge_index[1].astype(jnp.uint32)
    attr = edge_attr.astype(jnp.uint32)
    keys = ((dst << jnp.uint32(18)) | (src << jnp.uint32(4)) | attr)
    ks = jnp.sort(keys)
    bounds = (jnp.arange(NBLK + 1, dtype=jnp.uint32) * NB) << jnp.uint32(18)
    offs = jnp.searchsorted(ks, bounds).astype(jnp.int32)
    offs = jnp.pad(offs, (0, OFFS_PAD - (NBLK + 1)))
    ksp = jnp.pad(ks, (0, EPAD - N_EDGES))

    x3 = x.astype(jnp.int32).reshape(NRB, 1, RB)
    batch3 = batch.astype(jnp.int32).reshape(NRB, 1, RB)

    nt = node_table.astype(f32)
    et = edge_table.astype(f32)
    pad_cols = lambda m: jnp.pad(m, ((0, 0), (0, FP - F)))
    wd, ws, we, prebp = [], [], [], []
    for l in range(4):
        w2d = pre_W[l].astype(f32).transpose(1, 0, 2).reshape(225, F)
        wd.append(pad_cols(w2d[:75]))
        ws.append(pad_cols(w2d[75:150]))
        we.append(pad_cols(w2d[150:]))
        prebp.append(pad_cols(pre_b[l].astype(f32).reshape(1, F)))

    h = _embed(x3, nt)
    for l in range(4):
        a, bm, c = _project(h, wd[l], ws[l], et, ee_W[l].astype(f32),
                            ee_b[l].astype(f32).reshape(1, 75), we[l],
                            prebp[l])
        s1, s2, mn, mx, deg = _segreduce(bm, c, ksp, offs)
        pre, st = _finalize(h, a, s1, s2, mn, mx, deg,
                            post_W[l].astype(f32), post_b[l].astype(f32),
                            lin_W[l].astype(f32),
                            lin_b[l].astype(f32).reshape(1, 75))
        h = _bn(pre, st, bn_g[l].astype(f32).reshape(1, 75),
                bn_b[l].astype(f32).reshape(1, 75))

    return _pool(batch3, h, m1W.astype(f32), m1b.astype(f32).reshape(1, 50),
                 m2W.astype(f32), m2b.astype(f32).reshape(1, 25),
                 m3W.astype(f32), m3b.astype(f32).reshape(1, 10))


# speedup vs baseline: 47.5188x; 1.0097x over previous
import functools

import jax
import jax.numpy as jnp
import numpy as np
from jax import lax
from jax.experimental import pallas as pl
from jax.experimental.pallas import tpu as pltpu
from jax.experimental.pallas import tpu_sc as plsc

N_NODES = 10000
N_EDGES = 160000
N_GRAPHS = 256
NB = 32
NBLK = 313
NP = NBLK * NB
NWORK = 32
BPW = 10
K = 128
EPAD = N_EDGES + K
F = 375
FP = 384
NSL = FP // 16
RB = 1000
NRB = N_NODES // RB
NOFF_PAD = 10048

_DEG_HIST = np.array([0, 0, 1, 3, 10, 26, 60, 120, 211, 331, 473, 620, 744,
                      826, 862, 855, 806, 724, 621, 510, 403, 306, 224, 158,
                      107, 70, 44, 27, 16, 9, 5, 3, 1], dtype=np.float64)
AVG_DEG_LOG = float((np.log(np.arange(len(_DEG_HIST)) + 1.0) * _DEG_HIST).sum()
                    / _DEG_HIST.sum())
HI = lax.Precision.HIGHEST
f32 = jnp.float32


def _dot(a, b):
    return jnp.dot(a, b, preferred_element_type=f32, precision=HI)


def _embed_body(x_ref, nt_ref, h_ref):
    xv = x_ref[0, 0, :]
    oh = (xv[:, None] == lax.broadcasted_iota(jnp.int32, (RB, 128), 1)
          ).astype(f32)
    h_ref[...] = _dot(oh, nt_ref[...])


def _embed(x3, node_table):
    return pl.pallas_call(
        _embed_body,
        grid=(NRB,),
        in_specs=[pl.BlockSpec((1, 1, RB), lambda i: (i, 0, 0)),
                  pl.BlockSpec((128, 75), lambda i: (0, 0))],
        out_specs=pl.BlockSpec((RB, 75), lambda i: (i, 0)),
        out_shape=jax.ShapeDtypeStruct((N_NODES, 75), f32),
    )(x3, node_table)


def _project_body(h_ref, wd_ref, ws_ref, et_ref, eew_ref, eeb_ref, we_ref,
                  preb_ref, a_ref, b_ref, c_ref):
    h = h_ref[...]
    a_ref[...] = _dot(h, wd_ref[...])
    b_ref[...] = _dot(h, ws_ref[...])

    @pl.when(pl.program_id(0) == 0)
    def _():
        eenc = _dot(et_ref[...], eew_ref[...]) + eeb_ref[...]
        c_ref[...] = _dot(eenc, we_ref[...]) + preb_ref[...]


def _project(h, wd, ws, edge_table, eew, eeb, we, preb):
    return pl.pallas_call(
        _project_body,
        grid=(NRB,),
        in_specs=[pl.BlockSpec((RB, 75), lambda i: (i, 0)),
                  pl.BlockSpec((75, FP), lambda i: (0, 0)),
                  pl.BlockSpec((75, FP), lambda i: (0, 0)),
                  pl.BlockSpec((16, 50), lambda i: (0, 0)),
                  pl.BlockSpec((50, 75), lambda i: (0, 0)),
                  pl.BlockSpec((1, 75), lambda i: (0, 0)),
                  pl.BlockSpec((75, FP), lambda i: (0, 0)),
                  pl.BlockSpec((1, FP), lambda i: (0, 0))],
        out_specs=[pl.BlockSpec((RB, FP), lambda i: (i, 0)),
                   pl.BlockSpec((RB, FP), lambda i: (i, 0)),
                   pl.BlockSpec((16, FP), lambda i: (0, 0))],
        out_shape=[jax.ShapeDtypeStruct((N_NODES, FP), f32),
                   jax.ShapeDtypeStruct((N_NODES, FP), f32),
                   jax.ShapeDtypeStruct((16, FP), f32)],
    )(h, wd, ws, edge_table, eew, eeb, we, preb)


def _scalar_at(ref, i):
    return ref[pl.ds(i, 16)][0]


def _seg_body(b_hbm, c_hbm, keys_hbm, noff_hbm, s1_hbm, s2_hbm, mn_hbm,
              mx_hbm, deg_hbm, b_rows, kbuf, src_buf, attr_buf,
              c_loc, acc_s1, acc_s2, acc_mn, acc_mx, acc_dg, noff_loc, sem):
    wid = lax.axis_index("s") + 16 * lax.axis_index("c")
    pltpu.sync_copy(c_hbm, c_loc)

    zero16 = jnp.zeros((16,), f32)
    inf16 = jnp.full((16,), jnp.inf, f32)
    ninf16 = jnp.full((16,), -jnp.inf, f32)
    one16 = jnp.full((16,), 1.0, f32)

    def blk_loop(i, carry):
        blk = wid + NWORK * i

        @pl.when(blk < NBLK)
        def _():
            node0 = pl.multiple_of(blk * NB, 8)
            pltpu.sync_copy(noff_hbm.at[pl.ds(node0, 64)], noff_loc)
            e0 = _scalar_at(noff_loc, 0)
            e1 = _scalar_at(noff_loc, NB)

            def init_row(r, c2):
                for c in range(NSL):
                    sl = pl.ds(c * 16, 16)
                    acc_s1[r, sl] = zero16
                    acc_s2[r, sl] = zero16
                    acc_mn[r, sl] = inf16
                    acc_mx[r, sl] = ninf16
                cnt = (_scalar_at(noff_loc, r + 1)
                       - _scalar_at(noff_loc, r)).astype(f32)
                acc_dg[r, pl.ds(0, 16)] = one16 * cnt
                return c2
            lax.fori_loop(0, NB, init_row, 0)

            al0 = (e0 >> 3) << 3
            nch = (e1 - al0 + (K - 1)) >> 7

            def ch_loop(ci, n_cur):
                eb = pl.multiple_of(al0 + ci * K, 8)
                pltpu.sync_copy(keys_hbm.at[pl.ds(eb, K)], kbuf)
                for v in range(K // 16):
                    sl = pl.ds(v * 16, 16)
                    kv = kbuf[sl]
                    src_buf[sl] = ((kv >> jnp.uint32(4))
                                   & jnp.uint32(0x3FFF)).astype(jnp.int32)
                    attr_buf[sl] = (kv & jnp.uint32(0xF)).astype(jnp.int32)
                pltpu.async_copy(b_hbm.at[src_buf], b_rows, sem).wait()
                lo = jnp.maximum(e0, eb)
                hi = jnp.minimum(e1, eb + K)

                def n_step(k, st):
                    n, brk = st
                    nl = _scalar_at(noff_loc, n)
                    nr = _scalar_at(noff_loc, n + 1)
                    active = brk == 0
                    l = jnp.maximum(nl, lo)
                    h = jnp.minimum(nr, hi)

                    @pl.when(active)
                    def _():
                        regs0 = []
                        for c in range(NSL):
                            sl = pl.ds(c * 16, 16)
                            regs0.append(acc_mn[n, sl])
                            regs0.append(acc_mx[n, sl])

                        def e_loop(j, regs):
                            jl = j - eb
                            av = _scalar_at(attr_buf, jl)
                            out = []
                            for c in range(NSL):
                                sl = pl.ds(c * 16, 16)
                                t = b_rows[jl, sl] + c_loc[av, sl]
                                plsc.addupdate(acc_s1.at[n, sl], t)
                                plsc.addupdate(acc_s2.at[n, sl], t * t)
                                out.append(jnp.minimum(regs[2 * c], t))
                                out.append(jnp.maximum(regs[2 * c + 1], t))
                            return out
                        regs = lax.fori_loop(l, h, e_loop, regs0)
                        for c in range(NSL):
                            sl = pl.ds(c * 16, 16)
                            acc_mn[n, sl] = regs[2 * c]
                            acc_mx[n, sl] = regs[2 * c + 1]

                    n2 = jnp.where(active & (nr <= hi), n + 1, n)
                    brk2 = jnp.where(active & (nr >= hi), jnp.int32(1), brk)
                    return (n2, brk2)

                n_after, _ = lax.fori_loop(0, NB + 1, n_step,
                                           (n_cur, jnp.int32(0)))
                return n_after
            lax.fori_loop(0, nch, ch_loop, jnp.int32(0))

            pltpu.sync_copy(acc_s1, s1_hbm.at[pl.ds(node0, NB)])
            pltpu.sync_copy(acc_s2, s2_hbm.at[pl.ds(node0, NB)])
            pltpu.sync_copy(acc_mn, mn_hbm.at[pl.ds(node0, NB)])
            pltpu.sync_copy(acc_mx, mx_hbm.at[pl.ds(node0, NB)])
            pltpu.sync_copy(acc_dg, deg_hbm.at[pl.ds(node0, NB)])
        return carry

    lax.fori_loop(0, BPW, blk_loop, 0)


@functools.cache
def _build_segreduce():
  return pl.kernel(
    _seg_body,
    mesh=plsc.VectorSubcoreMesh(core_axis_name="c", subcore_axis_name="s"),
    out_type=[jax.ShapeDtypeStruct((NP, FP), f32),
              jax.ShapeDtypeStruct((NP, FP), f32),
              jax.ShapeDtypeStruct((NP, FP), f32),
              jax.ShapeDtypeStruct((NP, FP), f32),
              jax.ShapeDtypeStruct((NP, 16), f32)],
    scratch_types=[pltpu.VMEM((K, FP), f32),
                   pltpu.VMEM((K,), jnp.uint32),
                   pltpu.VMEM((K,), jnp.int32),
                   pltpu.VMEM((K + 16,), jnp.int32),
                   pltpu.VMEM((16, FP), f32),
                   pltpu.VMEM((NB, FP), f32),
                   pltpu.VMEM((NB, FP), f32),
                   pltpu.VMEM((NB, FP), f32),
                   pltpu.VMEM((NB, FP), f32),
                   pltpu.VMEM((NB, 16), f32),
                   pltpu.VMEM((64,), jnp.int32),
                   pltpu.SemaphoreType.DMA])


def _segreduce(bm, c, ksp, noff):
    return _build_segreduce()(bm, c, ksp, noff)


def _final_body(h_ref, a_ref, s1_ref, s2_ref, mn_ref, mx_ref, deg_ref,
                pw_ref, pb_ref, lw_ref, lb_ref, pre_ref, st_ref):
    deg_raw = deg_ref[:, 0:1]
    has = deg_raw > 0
    degc = jnp.maximum(deg_raw, 1.0)
    a = a_ref[:, :F]
    s1d = s1_ref[:, :F] / degc
    mean = jnp.where(has, a + s1d, 0.0)
    sqm = jnp.where(has, a * a + 2.0 * a * s1d + s2_ref[:, :F] / degc, 0.0)
    std = jnp.sqrt(jax.nn.relu(sqm - mean * mean) + 1e-5)
    mn = jnp.where(has, a + mn_ref[:, :F], 0.0)
    mx = jnp.where(has, a + mx_ref[:, :F], 0.0)
    amp = jnp.log(degc + 1.0) * (1.0 / AVG_DEG_LOG)
    iamp = 1.0 / amp
    h = h_ref[...]
    ys = []
    for t in range(5):
        sl = slice(t * 75, (t + 1) * 75)
        parts = [mean[:, sl], mn[:, sl], mx[:, sl], std[:, sl]]
        feats = jnp.concatenate(
            [h] + parts + [p * amp for p in parts] + [p * iamp for p in parts],
            axis=1)
        ys.append(_dot(feats, pw_ref[t]) + pb_ref[t, :][None, :])
    out75 = jnp.concatenate(ys, axis=1)
    pre = _dot(out75, lw_ref[...]) + lb_ref[...]
    pre_ref[...] = pre

    @pl.when(pl.program_id(0) == 0)
    def _():
        st_ref[...] = jnp.zeros((8, 128), f32)

    st_ref[0:1, :75] = st_ref[0:1, :75] + jnp.sum(pre, axis=0)[None, :]
    st_ref[1:2, :75] = st_ref[1:2, :75] + jnp.sum(pre * pre, axis=0)[None, :]


def _finalize(h, a, s1, s2, mn, mx, deg, pw, pb, lw, lb):
    return pl.pallas_call(
        _final_body,
        grid=(NRB,),
        in_specs=[pl.BlockSpec((RB, 75), lambda i: (i, 0)),
                  pl.BlockSpec((RB, FP), lambda i: (i, 0)),
                  pl.BlockSpec((RB, FP), lambda i: (i, 0)),
                  pl.BlockSpec((RB, FP), lambda i: (i, 0)),
                  pl.BlockSpec((RB, FP), lambda i: (i, 0)),
                  pl.BlockSpec((RB, FP), lambda i: (i, 0)),
                  pl.BlockSpec((RB, 16), lambda i: (i, 0)),
                  pl.BlockSpec((5, 975, 15), lambda i: (0, 0, 0)),
                  pl.BlockSpec((5, 15), lambda i: (0, 0)),
                  pl.BlockSpec((75, 75), lambda i: (0, 0)),
                  pl.BlockSpec((1, 75), lambda i: (0, 0))],
        out_specs=[pl.BlockSpec((RB, 75), lambda i: (i, 0)),
                   pl.BlockSpec((8, 128), lambda i: (0, 0))],
        out_shape=[jax.ShapeDtypeStruct((N_NODES, 75), f32),
                   jax.ShapeDtypeStruct((8, 128), f32)],
    )(h, a, s1, s2, mn, mx, deg, pw, pb, lw, lb)


def _bn_body(pre_ref, st_ref, g_ref, b_ref, h_ref):
    mu = st_ref[0:1, :75] * (1.0 / N_NODES)
    ex2 = st_ref[1:2, :75] * (1.0 / N_NODES)
    var = ex2 - mu * mu
    scale = g_ref[...] / jnp.sqrt(var + 1e-5)
    h_ref[...] = jax.nn.relu((pre_ref[...] - mu) * scale + b_ref[...])


def _bn(pre, st, g, b):
    return pl.pallas_call(
        _bn_body,
        grid=(NRB,),
        in_specs=[pl.BlockSpec((RB, 75), lambda i: (i, 0)),
                  pl.BlockSpec((8, 128), lambda i: (0, 0)),
                  pl.BlockSpec((1, 75), lambda i: (0, 0)),
                  pl.BlockSpec((1, 75), lambda i: (0, 0))],
        out_specs=pl.BlockSpec((RB, 75), lambda i: (i, 0)),
        out_shape=jax.ShapeDtypeStruct((N_NODES, 75), f32),
    )(pre, st, g, b)


def _pool_body(b3_ref, h_ref, w1_ref, b1_ref, w2_ref, b2_ref, w3_ref, b3b_ref,
               out_ref, g_ref):
    i = pl.program_id(0)

    @pl.when(i == 0)
    def _():
        g_ref[...] = jnp.zeros((N_GRAPHS, 75), f32)

    bv = b3_ref[0, 0, :]
    oh = (bv[:, None] == lax.broadcasted_iota(jnp.int32, (RB, N_GRAPHS), 1)
          ).astype(f32)
    g_ref[...] = g_ref[...] + lax.dot_general(
        oh, h_ref[...], (((0,), (0,)), ((), ())),
        preferred_element_type=f32, precision=HI)

    @pl.when(i == NRB - 1)
    def _():
        z = jax.nn.relu(_dot(g_ref[...], w1_ref[...]) + b1_ref[...])
        z = jax.nn.relu(_dot(z, w2_ref[...]) + b2_ref[...])
        out_ref[...] = _dot(z, w3_ref[...]) + b3b_ref[...]


def _pool(batch3, h, w1, b1, w2, b2, w3, b3):
    return pl.pallas_call(
        _pool_body,
        grid=(NRB,),
        in_specs=[pl.BlockSpec((1, 1, RB), lambda i: (i, 0, 0)),
                  pl.BlockSpec((RB, 75), lambda i: (i, 0)),
                  pl.BlockSpec((75, 50), lambda i: (0, 0)),
                  pl.BlockSpec((1, 50), lambda i: (0, 0)),
                  pl.BlockSpec((50, 25), lambda i: (0, 0)),
                  pl.BlockSpec((1, 25), lambda i: (0, 0)),
                  pl.BlockSpec((25, 10), lambda i: (0, 0)),
                  pl.BlockSpec((1, 10), lambda i: (0, 0))],
        out_specs=pl.BlockSpec((N_GRAPHS, 10), lambda i: (0, 0)),
        out_shape=jax.ShapeDtypeStruct((N_GRAPHS, 10), f32),
        scratch_shapes=[pltpu.VMEM((N_GRAPHS, 75), f32)],
    )(batch3, h, w1, b1, w2, b2, w3, b3)


def kernel(x, edge_index, edge_attr, batch, node_table, edge_table, ee_W,
           ee_b, pre_W, pre_b, post_W, post_b, lin_W, lin_b, bn_g, bn_b,
           m1W, m1b, m2W, m2b, m3W, m3b):
    src = edge_index[0].astype(jnp.uint32)
    dst = edge_index[1].astype(jnp.uint32)
    attr = edge_attr.astype(jnp.uint32)
    keys = ((dst << jnp.uint32(18)) | (src << jnp.uint32(4)) | attr)
    ks = jnp.sort(keys)
    bounds = jnp.arange(NOFF_PAD, dtype=jnp.uint32) << jnp.uint32(18)
    noff = jnp.searchsorted(ks, bounds).astype(jnp.int32)
    ksp = jnp.pad(ks, (0, EPAD - N_EDGES))

    x3 = x.astype(jnp.int32).reshape(NRB, 1, RB)
    batch3 = batch.astype(jnp.int32).reshape(NRB, 1, RB)

    nt = node_table.astype(f32)
    et = edge_table.astype(f32)
    pad_cols = lambda m: jnp.pad(m, ((0, 0), (0, FP - F)))
    wd, ws, we, prebp = [], [], [], []
    for l in range(4):
        w2d = pre_W[l].astype(f32).transpose(1, 0, 2).reshape(225, F)
        wd.append(pad_cols(w2d[:75]))
        ws.append(pad_cols(w2d[75:150]))
        we.append(pad_cols(w2d[150:]))
        prebp.append(pad_cols(pre_b[l].astype(f32).reshape(1, F)))

    h = _embed(x3, nt)
    for l in range(4):
        a, bm, c = _project(h, wd[l], ws[l], et, ee_W[l].astype(f32),
                            ee_b[l].astype(f32).reshape(1, 75), we[l],
                            prebp[l])
        s1, s2, mn, mx, deg = _segreduce(bm, c, ksp, noff)
        pre, st = _finalize(h, a, s1, s2, mn, mx, deg,
                            post_W[l].astype(f32), post_b[l].astype(f32),
                            lin_W[l].astype(f32),
                            lin_b[l].astype(f32).reshape(1, 75))
        h = _bn(pre, st, bn_g[l].astype(f32).reshape(1, 75),
                bn_b[l].astype(f32).reshape(1, 75))

    return _pool(batch3, h, m1W.astype(f32), m1b.astype(f32).reshape(1, 50),
                 m2W.astype(f32), m2b.astype(f32).reshape(1, 25),
                 m3W.astype(f32), m3b.astype(f32).reshape(1, 10))


# speedup vs baseline: 49.0334x; 1.0319x over previous
import functools

import jax
import jax.numpy as jnp
import numpy as np
from jax import lax
from jax.experimental import pallas as pl
from jax.experimental.pallas import tpu as pltpu
from jax.experimental.pallas import tpu_sc as plsc

N_NODES = 10000
N_EDGES = 160000
N_GRAPHS = 256
NB = 32
NBLK = 313
NP = NBLK * NB
NWORK = 32
BPW = 10
K = 64
EPAD = N_EDGES + K
F = 375
FP = 384
NSL = FP // 16
RB = 1000
NRB = N_NODES // RB
NOFF_PAD = 10048

_DEG_HIST = np.array([0, 0, 1, 3, 10, 26, 60, 120, 211, 331, 473, 620, 744,
                      826, 862, 855, 806, 724, 621, 510, 403, 306, 224, 158,
                      107, 70, 44, 27, 16, 9, 5, 3, 1], dtype=np.float64)
AVG_DEG_LOG = float((np.log(np.arange(len(_DEG_HIST)) + 1.0) * _DEG_HIST).sum()
                    / _DEG_HIST.sum())
HI = lax.Precision.HIGHEST
f32 = jnp.float32


def _dot(a, b):
    return jnp.dot(a, b, preferred_element_type=f32, precision=HI)


def _embed_body(x_ref, nt_ref, h_ref):
    xv = x_ref[0, 0, :]
    oh = (xv[:, None] == lax.broadcasted_iota(jnp.int32, (RB, 128), 1)
          ).astype(f32)
    h_ref[...] = _dot(oh, nt_ref[...])


def _embed(x3, node_table):
    return pl.pallas_call(
        _embed_body,
        grid=(NRB,),
        in_specs=[pl.BlockSpec((1, 1, RB), lambda i: (i, 0, 0)),
                  pl.BlockSpec((128, 75), lambda i: (0, 0))],
        out_specs=pl.BlockSpec((RB, 75), lambda i: (i, 0)),
        out_shape=jax.ShapeDtypeStruct((N_NODES, 75), f32),
    )(x3, node_table)


def _project_body(h_ref, wd_ref, ws_ref, et_ref, eew_ref, eeb_ref, we_ref,
                  preb_ref, a_ref, b_ref, c_ref):
    h = h_ref[...]
    a_ref[...] = _dot(h, wd_ref[...])
    b_ref[...] = _dot(h, ws_ref[...])

    @pl.when(pl.program_id(0) == 0)
    def _():
        eenc = _dot(et_ref[...], eew_ref[...]) + eeb_ref[...]
        c_ref[...] = _dot(eenc, we_ref[...]) + preb_ref[...]


def _project(h, wd, ws, edge_table, eew, eeb, we, preb):
    return pl.pallas_call(
        _project_body,
        grid=(NRB,),
        in_specs=[pl.BlockSpec((RB, 75), lambda i: (i, 0)),
                  pl.BlockSpec((75, FP), lambda i: (0, 0)),
                  pl.BlockSpec((75, FP), lambda i: (0, 0)),
                  pl.BlockSpec((16, 50), lambda i: (0, 0)),
                  pl.BlockSpec((50, 75), lambda i: (0, 0)),
                  pl.BlockSpec((1, 75), lambda i: (0, 0)),
                  pl.BlockSpec((75, FP), lambda i: (0, 0)),
                  pl.BlockSpec((1, FP), lambda i: (0, 0))],
        out_specs=[pl.BlockSpec((RB, FP), lambda i: (i, 0)),
                   pl.BlockSpec((RB, FP), lambda i: (i, 0)),
                   pl.BlockSpec((16, FP), lambda i: (0, 0))],
        out_shape=[jax.ShapeDtypeStruct((N_NODES, FP), f32),
                   jax.ShapeDtypeStruct((N_NODES, FP), f32),
                   jax.ShapeDtypeStruct((16, FP), f32)],
    )(h, wd, ws, edge_table, eew, eeb, we, preb)


def _scalar_at(ref, i):
    return ref[pl.ds(i, 16)][0]


def _seg_body(b_hbm, c_hbm, keys_hbm, noff_hbm, s1_hbm, s2_hbm, mn_hbm,
              mx_hbm, deg_hbm, br0, br1, kbuf, sb0, sb1, ab0, ab1,
              c_loc, acc_s1, acc_s2, acc_mn, acc_mx, acc_dg, noff_loc, sem):
    wid = lax.axis_index("s") + 16 * lax.axis_index("c")
    pltpu.sync_copy(c_hbm, c_loc)

    zero16 = jnp.zeros((16,), f32)
    inf16 = jnp.full((16,), jnp.inf, f32)
    ninf16 = jnp.full((16,), -jnp.inf, f32)
    one16 = jnp.full((16,), 1.0, f32)

    def blk_loop(i, carry):
        blk = wid + NWORK * i

        @pl.when(blk < NBLK)
        def _():
            node0 = pl.multiple_of(blk * NB, 8)
            pltpu.sync_copy(noff_hbm.at[pl.ds(node0, 64)], noff_loc)
            e0 = _scalar_at(noff_loc, 0)
            e1 = _scalar_at(noff_loc, NB)

            def init_row(r, c2):
                for c in range(NSL):
                    sl = pl.ds(c * 16, 16)
                    acc_s1[r, sl] = zero16
                    acc_s2[r, sl] = zero16
                    acc_mn[r, sl] = inf16
                    acc_mx[r, sl] = ninf16
                cnt = (_scalar_at(noff_loc, r + 1)
                       - _scalar_at(noff_loc, r)).astype(f32)
                acc_dg[r, pl.ds(0, 16)] = one16 * cnt
                return c2
            lax.fori_loop(0, NB, init_row, 0)

            al0 = (e0 >> 3) << 3
            nch = (e1 - al0 + (K - 1)) >> 6

            def issue(ci, sb, ab, br):
                eb = pl.multiple_of(al0 + ci * K, 8)
                pltpu.sync_copy(keys_hbm.at[pl.ds(eb, K)], kbuf)
                for v in range(K // 16):
                    sl = pl.ds(v * 16, 16)
                    kv = kbuf[sl]
                    sb[sl] = ((kv >> jnp.uint32(4))
                              & jnp.uint32(0x3FFF)).astype(jnp.int32)
                    ab[sl] = (kv & jnp.uint32(0xF)).astype(jnp.int32)
                pltpu.async_copy(b_hbm.at[sb], br, sem)

            def process(ci, br, ab, n_cur):
                eb = al0 + ci * K
                lo = jnp.maximum(e0, eb)
                hi = jnp.minimum(e1, eb + K)

                def n_step(k, st):
                    n, brk = st
                    nl = _scalar_at(noff_loc, n)
                    nr = _scalar_at(noff_loc, n + 1)
                    active = (brk == 0) & (lo < hi)
                    l = jnp.maximum(nl, lo)
                    h = jnp.minimum(nr, hi)

                    @pl.when(active)
                    def _():
                        regs0 = []
                        for c in range(NSL):
                            sl = pl.ds(c * 16, 16)
                            regs0.append(acc_mn[n, sl])
                            regs0.append(acc_mx[n, sl])

                        def e_loop(j, regs):
                            jl = j - eb
                            av = _scalar_at(ab, jl)
                            out = []
                            for c in range(NSL):
                                sl = pl.ds(c * 16, 16)
                                t = br[jl, sl] + c_loc[av, sl]
                                plsc.addupdate(acc_s1.at[n, sl], t)
                                plsc.addupdate(acc_s2.at[n, sl], t * t)
                                out.append(jnp.minimum(regs[2 * c], t))
                                out.append(jnp.maximum(regs[2 * c + 1], t))
                            return out
                        regs = lax.fori_loop(l, h, e_loop, regs0)
                        for c in range(NSL):
                            sl = pl.ds(c * 16, 16)
                            acc_mn[n, sl] = regs[2 * c]
                            acc_mx[n, sl] = regs[2 * c + 1]

                    n2 = jnp.where(active & (nr <= hi), n + 1, n)
                    brk2 = jnp.where(active & (nr >= hi), jnp.int32(1), brk)
                    return (n2, brk2)

                n_after, _ = lax.fori_loop(0, NB + 1, n_step,
                                           (n_cur, jnp.int32(0)))
                return n_after

            @pl.when(nch > 0)
            def _():
                issue(jnp.int32(0), sb0, ab0, br0)

            def pair_loop(p, n_cur):
                ci0 = 2 * p
                ci1 = ci0 + 1
                pltpu.make_async_copy(b_hbm.at[sb0], br0, sem).wait()

                @pl.when(ci1 < nch)
                def _():
                    issue(ci1, sb1, ab1, br1)
                n_cur = process(ci0, br0, ab0, n_cur)

                @pl.when(ci1 < nch)
                def _():
                    pltpu.make_async_copy(b_hbm.at[sb1], br1, sem).wait()

                    @pl.when(ci1 + 1 < nch)
                    def _():
                        issue(ci1 + 1, sb0, ab0, br0)
                n_cur = process(ci1, br1, ab1, n_cur)
                return n_cur
            lax.fori_loop(0, (nch + 1) >> 1, pair_loop, jnp.int32(0))

            pltpu.sync_copy(acc_s1, s1_hbm.at[pl.ds(node0, NB)])
            pltpu.sync_copy(acc_s2, s2_hbm.at[pl.ds(node0, NB)])
            pltpu.sync_copy(acc_mn, mn_hbm.at[pl.ds(node0, NB)])
            pltpu.sync_copy(acc_mx, mx_hbm.at[pl.ds(node0, NB)])
            pltpu.sync_copy(acc_dg, deg_hbm.at[pl.ds(node0, NB)])
        return carry

    lax.fori_loop(0, BPW, blk_loop, 0)


@functools.cache
def _build_segreduce():
  return pl.kernel(
    _seg_body,
    mesh=plsc.VectorSubcoreMesh(core_axis_name="c", subcore_axis_name="s"),
    out_type=[jax.ShapeDtypeStruct((NP, FP), f32),
              jax.ShapeDtypeStruct((NP, FP), f32),
              jax.ShapeDtypeStruct((NP, FP), f32),
              jax.ShapeDtypeStruct((NP, FP), f32),
              jax.ShapeDtypeStruct((NP, 16), f32)],
    scratch_types=[pltpu.VMEM((K, FP), f32),
                   pltpu.VMEM((K, FP), f32),
                   pltpu.VMEM((K,), jnp.uint32),
                   pltpu.VMEM((K,), jnp.int32),
                   pltpu.VMEM((K,), jnp.int32),
                   pltpu.VMEM((K + 16,), jnp.int32),
                   pltpu.VMEM((K + 16,), jnp.int32),
                   pltpu.VMEM((16, FP), f32),
                   pltpu.VMEM((NB, FP), f32),
                   pltpu.VMEM((NB, FP), f32),
                   pltpu.VMEM((NB, FP), f32),
                   pltpu.VMEM((NB, FP), f32),
                   pltpu.VMEM((NB, 16), f32),
                   pltpu.VMEM((64,), jnp.int32),
                   pltpu.SemaphoreType.DMA])


def _segreduce(bm, c, ksp, noff):
    return _build_segreduce()(bm, c, ksp, noff)


def _final_body(h_ref, a_ref, s1_ref, s2_ref, mn_ref, mx_ref, deg_ref,
                pw_ref, pb_ref, lw_ref, lb_ref, pre_ref, st_ref):
    deg_raw = deg_ref[:, 0:1]
    has = deg_raw > 0
    degc = jnp.maximum(deg_raw, 1.0)
    a = a_ref[:, :F]
    s1d = s1_ref[:, :F] / degc
    mean = jnp.where(has, a + s1d, 0.0)
    sqm = jnp.where(has, a * a + 2.0 * a * s1d + s2_ref[:, :F] / degc, 0.0)
    std = jnp.sqrt(jax.nn.relu(sqm - mean * mean) + 1e-5)
    mn = jnp.where(has, a + mn_ref[:, :F], 0.0)
    mx = jnp.where(has, a + mx_ref[:, :F], 0.0)
    amp = jnp.log(degc + 1.0) * (1.0 / AVG_DEG_LOG)
    iamp = 1.0 / amp
    h = h_ref[...]
    ys = []
    for t in range(5):
        sl = slice(t * 75, (t + 1) * 75)
        parts = [mean[:, sl], mn[:, sl], mx[:, sl], std[:, sl]]
        feats = jnp.concatenate(
            [h] + parts + [p * amp for p in parts] + [p * iamp for p in parts],
            axis=1)
        ys.append(_dot(feats, pw_ref[t]) + pb_ref[t, :][None, :])
    out75 = jnp.concatenate(ys, axis=1)
    pre = _dot(out75, lw_ref[...]) + lb_ref[...]
    pre_ref[...] = pre

    @pl.when(pl.program_id(0) == 0)
    def _():
        st_ref[...] = jnp.zeros((8, 128), f32)

    st_ref[0:1, :75] = st_ref[0:1, :75] + jnp.sum(pre, axis=0)[None, :]
    st_ref[1:2, :75] = st_ref[1:2, :75] + jnp.sum(pre * pre, axis=0)[None, :]


def _finalize(h, a, s1, s2, mn, mx, deg, pw, pb, lw, lb):
    return pl.pallas_call(
        _final_body,
        grid=(NRB,),
        in_specs=[pl.BlockSpec((RB, 75), lambda i: (i, 0)),
                  pl.BlockSpec((RB, FP), lambda i: (i, 0)),
                  pl.BlockSpec((RB, FP), lambda i: (i, 0)),
                  pl.BlockSpec((RB, FP), lambda i: (i, 0)),
                  pl.BlockSpec((RB, FP), lambda i: (i, 0)),
                  pl.BlockSpec((RB, FP), lambda i: (i, 0)),
                  pl.BlockSpec((RB, 16), lambda i: (i, 0)),
                  pl.BlockSpec((5, 975, 15), lambda i: (0, 0, 0)),
                  pl.BlockSpec((5, 15), lambda i: (0, 0)),
                  pl.BlockSpec((75, 75), lambda i: (0, 0)),
                  pl.BlockSpec((1, 75), lambda i: (0, 0))],
        out_specs=[pl.BlockSpec((RB, 75), lambda i: (i, 0)),
                   pl.BlockSpec((8, 128), lambda i: (0, 0))],
        out_shape=[jax.ShapeDtypeStruct((N_NODES, 75), f32),
                   jax.ShapeDtypeStruct((8, 128), f32)],
    )(h, a, s1, s2, mn, mx, deg, pw, pb, lw, lb)


def _bn_body(pre_ref, st_ref, g_ref, b_ref, h_ref):
    mu = st_ref[0:1, :75] * (1.0 / N_NODES)
    ex2 = st_ref[1:2, :75] * (1.0 / N_NODES)
    var = ex2 - mu * mu
    scale = g_ref[...] / jnp.sqrt(var + 1e-5)
    h_ref[...] = jax.nn.relu((pre_ref[...] - mu) * scale + b_ref[...])


def _bn(pre, st, g, b):
    return pl.pallas_call(
        _bn_body,
        grid=(NRB,),
        in_specs=[pl.BlockSpec((RB, 75), lambda i: (i, 0)),
                  pl.BlockSpec((8, 128), lambda i: (0, 0)),
                  pl.BlockSpec((1, 75), lambda i: (0, 0)),
                  pl.BlockSpec((1, 75), lambda i: (0, 0))],
        out_specs=pl.BlockSpec((RB, 75), lambda i: (i, 0)),
        out_shape=jax.ShapeDtypeStruct((N_NODES, 75), f32),
    )(pre, st, g, b)


def _pool_body(b3_ref, h_ref, w1_ref, b1_ref, w2_ref, b2_ref, w3_ref, b3b_ref,
               out_ref, g_ref):
    i = pl.program_id(0)

    @pl.when(i == 0)
    def _():
        g_ref[...] = jnp.zeros((N_GRAPHS, 75), f32)

    bv = b3_ref[0, 0, :]
    oh = (bv[:, None] == lax.broadcasted_iota(jnp.int32, (RB, N_GRAPHS), 1)
          ).astype(f32)
    g_ref[...] = g_ref[...] + lax.dot_general(
        oh, h_ref[...], (((0,), (0,)), ((), ())),
        preferred_element_type=f32, precision=HI)

    @pl.when(i == NRB - 1)
    def _():
        z = jax.nn.relu(_dot(g_ref[...], w1_ref[...]) + b1_ref[...])
        z = jax.nn.relu(_dot(z, w2_ref[...]) + b2_ref[...])
        out_ref[...] = _dot(z, w3_ref[...]) + b3b_ref[...]


def _pool(batch3, h, w1, b1, w2, b2, w3, b3):
    return pl.pallas_call(
        _pool_body,
        grid=(NRB,),
        in_specs=[pl.BlockSpec((1, 1, RB), lambda i: (i, 0, 0)),
                  pl.BlockSpec((RB, 75), lambda i: (i, 0)),
                  pl.BlockSpec((75, 50), lambda i: (0, 0)),
                  pl.BlockSpec((1, 50), lambda i: (0, 0)),
                  pl.BlockSpec((50, 25), lambda i: (0, 0)),
                  pl.BlockSpec((1, 25), lambda i: (0, 0)),
                  pl.BlockSpec((25, 10), lambda i: (0, 0)),
                  pl.BlockSpec((1, 10), lambda i: (0, 0))],
        out_specs=pl.BlockSpec((N_GRAPHS, 10), lambda i: (0, 0)),
        out_shape=jax.ShapeDtypeStruct((N_GRAPHS, 10), f32),
        scratch_shapes=[pltpu.VMEM((N_GRAPHS, 75), f32)],
    )(batch3, h, w1, b1, w2, b2, w3, b3)


def kernel(x, edge_index, edge_attr, batch, node_table, edge_table, ee_W,
           ee_b, pre_W, pre_b, post_W, post_b, lin_W, lin_b, bn_g, bn_b,
           m1W, m1b, m2W, m2b, m3W, m3b):
    src = edge_index[0].astype(jnp.uint32)
    dst = edge_index[1].astype(jnp.uint32)
    attr = edge_attr.astype(jnp.uint32)
    keys = ((dst << jnp.uint32(18)) | (src << jnp.uint32(4)) | attr)
    ks = jnp.sort(keys)
    bounds = jnp.arange(NOFF_PAD, dtype=jnp.uint32) << jnp.uint32(18)
    noff = jnp.searchsorted(ks, bounds).astype(jnp.int32)
    ksp = jnp.pad(ks, (0, EPAD - N_EDGES))

    x3 = x.astype(jnp.int32).reshape(NRB, 1, RB)
    batch3 = batch.astype(jnp.int32).reshape(NRB, 1, RB)

    nt = node_table.astype(f32)
    et = edge_table.astype(f32)
    pad_cols = lambda m: jnp.pad(m, ((0, 0), (0, FP - F)))
    wd, ws, we, prebp = [], [], [], []
    for l in range(4):
        w2d = pre_W[l].astype(f32).transpose(1, 0, 2).reshape(225, F)
        wd.append(pad_cols(w2d[:75]))
        ws.append(pad_cols(w2d[75:150]))
        we.append(pad_cols(w2d[150:]))
        prebp.append(pad_cols(pre_b[l].astype(f32).reshape(1, F)))

    h = _embed(x3, nt)
    for l in range(4):
        a, bm, c = _project(h, wd[l], ws[l], et, ee_W[l].astype(f32),
                            ee_b[l].astype(f32).reshape(1, 75), we[l],
                            prebp[l])
        s1, s2, mn, mx, deg = _segreduce(bm, c, ksp, noff)
        pre, st = _finalize(h, a, s1, s2, mn, mx, deg,
                            post_W[l].astype(f32), post_b[l].astype(f32),
                            lin_W[l].astype(f32),
                            lin_b[l].astype(f32).reshape(1, 75))
        h = _bn(pre, st, bn_g[l].astype(f32).reshape(1, 75),
                bn_b[l].astype(f32).reshape(1, 75))

    return _pool(batch3, h, m1W.astype(f32), m1b.astype(f32).reshape(1, 50),
                 m2W.astype(f32), m2b.astype(f32).reshape(1, 25),
                 m3W.astype(f32), m3b.astype(f32).reshape(1, 10))
